# Initial kernel scaffold; baseline (speedup 1.0000x reference)
import jax
import jax.numpy as jnp
from jax.experimental import pallas as pl


def kernel(pred_logits, pred_boxes, tgt_ids, tgt_boxes):
    raise NotImplementedError("write your pallas kernel here")


# speedup vs baseline: 2.4577x; 2.4577x over previous
import jax
import jax.numpy as jnp
from jax.experimental import pallas as pl
from jax.experimental.pallas import tpu as pltpu

ALPHA = 0.25
W_CLASS = 2.0
W_BBOX = 5.0
W_GIOU = 2.0
EPS_LOG = 1e-8
EPS_DIV = 1e-6

_QM = 288


def _cost_kernel(logits_ref, pb_ref, tb_ref, tid_ref, out_ref):
    qm, t = out_ref.shape

    s = jax.nn.sigmoid(logits_ref[...])
    one_m = 1.0 - s
    neg = (1.0 - ALPHA) * (s * s) * (-jnp.log(one_m + EPS_LOG))
    pos = ALPHA * (one_m * one_m) * (-jnp.log(s + EPS_LOG))
    diff = pos - neg

    idx = jnp.broadcast_to(tid_ref[0:1, :] - 1, (qm, t))
    cost_class = jnp.take_along_axis(diff, idx, axis=1)

    cxq = pb_ref[:, 0:1]
    cyq = pb_ref[:, 1:2]
    wq = pb_ref[:, 2:3]
    hq = pb_ref[:, 3:4]
    cxt = tb_ref[0:1, :]
    cyt = tb_ref[1:2, :]
    wt = tb_ref[2:3, :]
    ht = tb_ref[3:4, :]

    cost_bbox = (jnp.abs(cxq - cxt) + jnp.abs(cyq - cyt)
                 + jnp.abs(wq - wt) + jnp.abs(hq - ht))

    x0q = cxq - 0.5 * wq
    y0q = cyq - 0.5 * hq
    x1q = cxq + 0.5 * wq
    y1q = cyq + 0.5 * hq
    x0t = cxt - 0.5 * wt
    y0t = cyt - 0.5 * ht
    x1t = cxt + 0.5 * wt
    y1t = cyt + 0.5 * ht
    areaq = (x1q - x0q) * (y1q - y0q)
    areat = (x1t - x0t) * (y1t - y0t)

    iw = jnp.maximum(jnp.minimum(x1q, x1t) - jnp.maximum(x0q, x0t), 0.0)
    ih = jnp.maximum(jnp.minimum(y1q, y1t) - jnp.maximum(y0q, y0t), 0.0)
    inter = iw * ih
    union = areaq + areat - inter
    iou = inter / jnp.maximum(union, EPS_DIV)
    ew = jnp.maximum(jnp.maximum(x1q, x1t) - jnp.minimum(x0q, x0t), 0.0)
    eh = jnp.maximum(jnp.maximum(y1q, y1t) - jnp.minimum(y0q, y0t), 0.0)
    encl = ew * eh
    giou = iou - (encl - union) / jnp.maximum(encl, EPS_DIV)

    out_ref[...] = (W_CLASS * cost_class + W_BBOX * cost_bbox - W_GIOU * giou)


def kernel(pred_logits, pred_boxes, tgt_ids, tgt_boxes):
    B, Q, C = pred_logits.shape
    T = tgt_ids.shape[0]
    BQ = B * Q

    logits = jnp.pad(pred_logits.reshape(BQ, C), ((0, 0), (0, 128 - C)))
    pb = pred_boxes.reshape(BQ, 4)
    tb = jnp.pad(tgt_boxes.T, ((0, 4), (0, 0)))
    tid = tgt_ids.astype(jnp.int32).reshape(1, T)

    out = pl.pallas_call(
        _cost_kernel,
        out_shape=jax.ShapeDtypeStruct((BQ, T), jnp.float32),
        grid=(BQ // _QM,),
        in_specs=[
            pl.BlockSpec((_QM, 128), lambda i: (i, 0)),
            pl.BlockSpec((_QM, 4), lambda i: (i, 0)),
            pl.BlockSpec((8, T), lambda i: (0, 0)),
            pl.BlockSpec((1, T), lambda i: (0, 0)),
        ],
        out_specs=pl.BlockSpec((_QM, T), lambda i: (i, 0)),
        compiler_params=pltpu.CompilerParams(
            dimension_semantics=("arbitrary",),
            vmem_limit_bytes=64 * 1024 * 1024,
        ),
        name="hungarian_cost_matrix",
    )(logits, pb, tb, tid)
    return out.reshape(B, Q, T)


# speedup vs baseline: 3.1841x; 1.2956x over previous
import jax
import jax.numpy as jnp
from jax.experimental import pallas as pl
from jax.experimental.pallas import tpu as pltpu

ALPHA = 0.25
W_CLASS = 2.0
W_BBOX = 5.0
W_GIOU = 2.0
EPS_LOG = 1e-8
EPS_DIV = 1e-6

_CHUNK = 128


def _cost_kernel(logits_ref, pb_ref, tb_ref, tid_ref, out_ref):
    q = out_ref.shape[1]
    t = out_ref.shape[2]

    idm1 = tid_ref[0:1, :] - 1
    cxt = tb_ref[0:1, :]
    cyt = tb_ref[1:2, :]
    wt = tb_ref[2:3, :]
    ht = tb_ref[3:4, :]
    x0t = cxt - 0.5 * wt
    y0t = cyt - 0.5 * ht
    x1t = cxt + 0.5 * wt
    y1t = cyt + 0.5 * ht
    areat = (x1t - x0t) * (y1t - y0t)

    def do_chunk(rows, m):
        s = jax.nn.sigmoid(logits_ref[0, rows, :])
        one_m = 1.0 - s
        neg = (1.0 - ALPHA) * (s * s) * (-jnp.log(one_m + EPS_LOG))
        pos = ALPHA * (one_m * one_m) * (-jnp.log(s + EPS_LOG))
        diff = pos - neg

        idx = jnp.broadcast_to(idm1, (m, t))
        cost_class = jnp.take_along_axis(diff, idx, axis=1)

        cxq = pb_ref[0, rows, 0:1]
        cyq = pb_ref[0, rows, 1:2]
        wq = pb_ref[0, rows, 2:3]
        hq = pb_ref[0, rows, 3:4]

        cost_bbox = (jnp.abs(cxq - cxt) + jnp.abs(cyq - cyt)
                     + jnp.abs(wq - wt) + jnp.abs(hq - ht))

        x0q = cxq - 0.5 * wq
        y0q = cyq - 0.5 * hq
        x1q = cxq + 0.5 * wq
        y1q = cyq + 0.5 * hq
        areaq = (x1q - x0q) * (y1q - y0q)

        iw = jnp.maximum(jnp.minimum(x1q, x1t) - jnp.maximum(x0q, x0t), 0.0)
        ih = jnp.maximum(jnp.minimum(y1q, y1t) - jnp.maximum(y0q, y0t), 0.0)
        inter = iw * ih
        union = areaq + areat - inter
        iou = inter / jnp.maximum(union, EPS_DIV)
        ew = jnp.maximum(jnp.maximum(x1q, x1t) - jnp.minimum(x0q, x0t), 0.0)
        eh = jnp.maximum(jnp.maximum(y1q, y1t) - jnp.minimum(y0q, y0t), 0.0)
        encl = ew * eh
        giou = iou - (encl - union) / jnp.maximum(encl, EPS_DIV)

        out_ref[0, rows, :] = (W_CLASS * cost_class + W_BBOX * cost_bbox
                               - W_GIOU * giou)

    n_full = q // _CHUNK

    def body(i, carry):
        a = pl.multiple_of(i * _CHUNK, _CHUNK)
        do_chunk(pl.ds(a, _CHUNK), _CHUNK)
        return carry

    jax.lax.fori_loop(0, n_full, body, 0)
    if q % _CHUNK:
        do_chunk(slice(n_full * _CHUNK, q), q - n_full * _CHUNK)


def kernel(pred_logits, pred_boxes, tgt_ids, tgt_boxes):
    B, Q, C = pred_logits.shape
    T = tgt_ids.shape[0]

    logits = jnp.pad(pred_logits, ((0, 0), (0, 0), (0, 128 - C)))
    tb = jnp.pad(tgt_boxes.T, ((0, 4), (0, 0)))
    tid = tgt_ids.astype(jnp.int32).reshape(1, T)

    return pl.pallas_call(
        _cost_kernel,
        out_shape=jax.ShapeDtypeStruct((B, Q, T), jnp.float32),
        grid=(B,),
        in_specs=[
            pl.BlockSpec((1, Q, 128), lambda i: (i, 0, 0)),
            pl.BlockSpec((1, Q, 4), lambda i: (i, 0, 0)),
            pl.BlockSpec((8, T), lambda i: (0, 0)),
            pl.BlockSpec((1, T), lambda i: (0, 0)),
        ],
        out_specs=pl.BlockSpec((1, Q, T), lambda i: (i, 0, 0)),
        compiler_params=pltpu.CompilerParams(
            dimension_semantics=("arbitrary",),
            vmem_limit_bytes=56 * 1024 * 1024,
        ),
        name="hungarian_cost_matrix",
    )(logits, pred_boxes, tb, tid)


# speedup vs baseline: 3.2662x; 1.0258x over previous
import jax
import jax.numpy as jnp
from jax.experimental import pallas as pl
from jax.experimental.pallas import tpu as pltpu

ALPHA = 0.25
W_CLASS = 2.0
W_BBOX = 5.0
W_GIOU = 2.0
EPS_LOG = 1e-8
EPS_DIV = 1e-6

_CHUNK = 128


def _cost_kernel(logits_ref, pb_ref, tb_ref, tid_ref, out_ref):
    q = out_ref.shape[1]
    t = out_ref.shape[2]

    idm1 = tid_ref[0:1, :] - 1
    cxt = tb_ref[0:1, :]
    cyt = tb_ref[1:2, :]
    wt = tb_ref[2:3, :]
    ht = tb_ref[3:4, :]
    x0t = cxt - 0.5 * wt
    y0t = cyt - 0.5 * ht
    x1t = cxt + 0.5 * wt
    y1t = cyt + 0.5 * ht
    areat = (x1t - x0t) * (y1t - y0t)

    def do_chunk(rows, m):
        s = jax.nn.sigmoid(logits_ref[0, rows, :])
        one_m = 1.0 - s
        neg = (1.0 - ALPHA) * (s * s) * (-jnp.log(one_m + EPS_LOG))
        pos = ALPHA * (one_m * one_m) * (-jnp.log(s + EPS_LOG))
        diff = W_CLASS * (pos - neg)

        idx = jnp.broadcast_to(idm1, (m, t))
        cost_class = jnp.take_along_axis(diff, idx, axis=1)

        cxq = pb_ref[0, rows, 0:1]
        cyq = pb_ref[0, rows, 1:2]
        wq = pb_ref[0, rows, 2:3]
        hq = pb_ref[0, rows, 3:4]

        cost_bbox = (jnp.abs(cxq - cxt) + jnp.abs(cyq - cyt)
                     + jnp.abs(wq - wt) + jnp.abs(hq - ht))

        x0q = cxq - 0.5 * wq
        y0q = cyq - 0.5 * hq
        x1q = cxq + 0.5 * wq
        y1q = cyq + 0.5 * hq
        areaq = (x1q - x0q) * (y1q - y0q)

        xlo = jnp.maximum(x0q, x0t)
        xhi = jnp.minimum(x1q, x1t)
        ylo = jnp.maximum(y0q, y0t)
        yhi = jnp.minimum(y1q, y1t)
        iw = jnp.maximum(xhi - xlo, 0.0)
        ih = jnp.maximum(yhi - ylo, 0.0)
        inter = iw * ih
        union = areaq + areat - inter
        iou = inter / jnp.maximum(union, EPS_DIV)
        ew = jnp.maximum(x1q, x1t) - jnp.minimum(x0q, x0t)
        eh = jnp.maximum(y1q, y1t) - jnp.minimum(y0q, y0t)
        encl = ew * eh
        giou = iou - (encl - union) / jnp.maximum(encl, EPS_DIV)

        out_ref[0, rows, :] = (cost_class + W_BBOX * cost_bbox
                               - W_GIOU * giou)

    n_full = q // _CHUNK

    def body(i, carry):
        a = pl.multiple_of(i * _CHUNK, _CHUNK)
        do_chunk(pl.ds(a, _CHUNK), _CHUNK)
        return carry

    jax.lax.fori_loop(0, n_full, body, 0)
    if q % _CHUNK:
        do_chunk(slice(n_full * _CHUNK, q), q - n_full * _CHUNK)


def kernel(pred_logits, pred_boxes, tgt_ids, tgt_boxes):
    B, Q, C = pred_logits.shape
    T = tgt_ids.shape[0]

    logits = jnp.pad(pred_logits, ((0, 0), (0, 0), (0, 128 - C)))
    tb = jnp.pad(tgt_boxes.T, ((0, 4), (0, 0)))
    tid = tgt_ids.astype(jnp.int32).reshape(1, T)

    return pl.pallas_call(
        _cost_kernel,
        out_shape=jax.ShapeDtypeStruct((B, Q, T), jnp.float32),
        grid=(B,),
        in_specs=[
            pl.BlockSpec((1, Q, 128), lambda i: (i, 0, 0)),
            pl.BlockSpec((1, Q, 4), lambda i: (i, 0, 0)),
            pl.BlockSpec((8, T), lambda i: (0, 0)),
            pl.BlockSpec((1, T), lambda i: (0, 0)),
        ],
        out_specs=pl.BlockSpec((1, Q, T), lambda i: (i, 0, 0)),
        compiler_params=pltpu.CompilerParams(
            dimension_semantics=("arbitrary",),
            vmem_limit_bytes=56 * 1024 * 1024,
        ),
        name="hungarian_cost_matrix",
    )(logits, pred_boxes, tb, tid)
